# Initial kernel scaffold; baseline (speedup 1.0000x reference)
import jax
import jax.numpy as jnp
from jax.experimental import pallas as pl


def kernel(x, edge_index, W1, b1, g1, be1, W2, b2, g2, be2, W3, b3):
    raise NotImplementedError("write your pallas kernel here")


# speedup vs baseline: 27.9686x; 27.9686x over previous
import functools

import jax
import jax.numpy as jnp
from jax import lax
from jax.experimental import pallas as pl
from jax.experimental.pallas import tpu as pltpu
from jax.experimental.pallas import tpu_sc as plsc

N = 10000
F_IN = 128
H1 = 32
H2 = 16
C = 40
CPAD = 48
E = 320000

NCORE = 2
NSUB = 16
NWORK = NCORE * NSUB
B = 128
WPT = 80
EPAD = NWORK * WPT * B
NPAD = 10240
RPT = NPAD // NSUB

_BN_SCALE = 1.0 / (1.0 + 1e-5) ** 0.5


def _sc_mesh():
    return plsc.VectorSubcoreMesh(core_axis_name="c", subcore_axis_name="s")


_SC_PARAMS = pltpu.CompilerParams(use_tc_tiling_on_sc=False)


def _make_degree_kernel():
    @functools.partial(
        pl.kernel,
        mesh=_sc_mesh(),
        out_type=(
            jax.ShapeDtypeStruct((NPAD,), jnp.float32),
            jax.ShapeDtypeStruct((NPAD,), jnp.float32),
        ),
        scratch_types=[
            pltpu.VMEM((WPT, B), jnp.int32),
            pltpu.VMEM((B,), jnp.float32),
            pltpu.VMEM_SHARED((NPAD,), jnp.float32),
        ],
        compiler_params=_SC_PARAMS,
    )
    def deg_kernel(col_hbm, zero_hbm, outa, outb, colbuf, ones, acc):
        core = lax.axis_index("c")
        sub = lax.axis_index("s")
        wid = core * NSUB + sub
        r0 = sub * RPT
        pltpu.sync_copy(zero_hbm.at[pl.ds(r0, RPT)], acc.at[pl.ds(r0, RPT)])
        pltpu.sync_copy(col_hbm.at[pl.ds(wid * WPT, WPT)], colbuf)
        for i in range(B // 16):
            ones[pl.ds(i * 16, 16)] = jnp.ones((16,), jnp.float32)
        plsc.subcore_barrier()

        def body(w, carry):
            pltpu.sync_copy(ones, acc.at[colbuf.at[w]], add=True)
            return carry

        lax.fori_loop(0, WPT, body, 0)
        plsc.subcore_barrier()

        @pl.when(core == 0)
        def _():
            pltpu.sync_copy(acc.at[pl.ds(r0, RPT)], outa.at[pl.ds(r0, RPT)])

        @pl.when(core == 1)
        def _():
            pltpu.sync_copy(acc.at[pl.ds(r0, RPT)], outb.at[pl.ds(r0, RPT)])

    return deg_kernel


def _make_gather_scatter(h):
    @functools.partial(
        pl.kernel,
        mesh=_sc_mesh(),
        out_type=(
            jax.ShapeDtypeStruct((NPAD, h), jnp.float32),
            jax.ShapeDtypeStruct((NPAD, h), jnp.float32),
        ),
        scratch_types=[
            pltpu.VMEM((WPT, B), jnp.int32),
            pltpu.VMEM((WPT, B), jnp.int32),
            pltpu.VMEM((B, h), jnp.float32),
            pltpu.VMEM_SHARED((NPAD, h), jnp.float32),
            pltpu.SemaphoreType.DMA,
        ],
        compiler_params=_SC_PARAMS,
    )
    def gs_kernel(z_hbm, row_hbm, col_hbm, zero_hbm, outa, outb,
                  rowbuf, colbuf, msg, acc, sem):
        core = lax.axis_index("c")
        sub = lax.axis_index("s")
        wid = core * NSUB + sub
        r0 = sub * RPT
        pltpu.sync_copy(zero_hbm.at[pl.ds(r0, RPT)], acc.at[pl.ds(r0, RPT)])
        pltpu.sync_copy(row_hbm.at[pl.ds(wid * WPT, WPT)], rowbuf)
        pltpu.sync_copy(col_hbm.at[pl.ds(wid * WPT, WPT)], colbuf)
        plsc.subcore_barrier()

        def body(w, carry):
            pltpu.async_copy(z_hbm.at[rowbuf.at[w]], msg, sem).wait()
            pltpu.sync_copy(msg, acc.at[colbuf.at[w]], add=True)
            return carry

        lax.fori_loop(0, WPT, body, 0)
        plsc.subcore_barrier()

        @pl.when(core == 0)
        def _():
            pltpu.sync_copy(acc.at[pl.ds(r0, RPT)], outa.at[pl.ds(r0, RPT)])

        @pl.when(core == 1)
        def _():
            pltpu.sync_copy(acc.at[pl.ds(r0, RPT)], outb.at[pl.ds(r0, RPT)])

    return gs_kernel


_deg_kernel = _make_degree_kernel()
_gs32 = _make_gather_scatter(H1)
_gs16 = _make_gather_scatter(H2)
_gs48 = _make_gather_scatter(CPAD)


def _tc_first(x_ref, w_ref, da_ref, db_ref, o_ref):
    dinv = lax.rsqrt(da_ref[...] + db_ref[...] + 1.0)
    xw = jnp.dot(x_ref[...], w_ref[...], preferred_element_type=jnp.float32)
    o_ref[...] = xw * dinv


def _tc_mid(sa_ref, sb_ref, z_ref, da_ref, db_ref, b_ref, g_ref, be_ref,
            w_ref, o_ref):
    dinv = lax.rsqrt(da_ref[...] + db_ref[...] + 1.0)
    conv = dinv * (sa_ref[...] + sb_ref[...] + z_ref[...]) + b_ref[...]
    h = jnp.maximum(conv * (g_ref[...] * _BN_SCALE) + be_ref[...], 0.0)
    o_ref[...] = jnp.dot(h, w_ref[...], preferred_element_type=jnp.float32) * dinv


def _tc_last(sa_ref, sb_ref, z_ref, da_ref, db_ref, b_ref, o_ref):
    dinv = lax.rsqrt(da_ref[...] + db_ref[...] + 1.0)
    o_ref[...] = jnp.maximum(dinv * (sa_ref[...] + sb_ref[...] + z_ref[...])
                             + b_ref[...], 0.0)


def _pc(fn, out_shape):
    return pl.pallas_call(fn, out_shape=out_shape)


def kernel(x, edge_index, W1, b1, g1, be1, W2, b2, g2, be2, W3, b3):
    f32 = jnp.float32
    row = edge_index[0]
    col = edge_index[1]
    pad = EPAD - E
    fill_row = (jnp.arange(pad, dtype=jnp.int32) * 41) % N
    fill_col = N + (jnp.arange(pad, dtype=jnp.int32) % 128)
    rowp = jnp.concatenate([row, fill_row]).reshape(NWORK * WPT, B)
    colp = jnp.concatenate([col, fill_col]).reshape(NWORK * WPT, B)

    xp = jnp.concatenate([x, jnp.zeros((NPAD - N, F_IN), f32)], axis=0)
    W3p = jnp.concatenate([W3, jnp.zeros((H2, CPAD - C), f32)], axis=1)
    b3p = jnp.concatenate([b3, jnp.zeros((CPAD - C,), f32)])

    zero1 = jnp.zeros((NPAD,), f32)
    zero32 = jnp.zeros((NPAD, H1), f32)
    zero16 = jnp.zeros((NPAD, H2), f32)
    zero48 = jnp.zeros((NPAD, CPAD), f32)

    dega, degb = _deg_kernel(colp, zero1)
    da = dega.reshape(NPAD, 1)
    db = degb.reshape(NPAD, 1)

    z1 = _pc(_tc_first, jax.ShapeDtypeStruct((NPAD, H1), f32))(xp, W1, da, db)
    s1a, s1b = _gs32(z1, rowp, colp, zero32)

    z2 = _pc(_tc_mid, jax.ShapeDtypeStruct((NPAD, H2), f32))(
        s1a, s1b, z1, da, db, b1.reshape(1, H1), g1.reshape(1, H1),
        be1.reshape(1, H1), W2)
    s2a, s2b = _gs16(z2, rowp, colp, zero16)

    z3 = _pc(_tc_mid, jax.ShapeDtypeStruct((NPAD, CPAD), f32))(
        s2a, s2b, z2, da, db, b2.reshape(1, H2), g2.reshape(1, H2),
        be2.reshape(1, H2), W3p)
    s3a, s3b = _gs48(z3, rowp, colp, zero48)

    out = _pc(_tc_last, jax.ShapeDtypeStruct((NPAD, CPAD), f32))(
        s3a, s3b, z3, da, db, b3p.reshape(1, CPAD))
    return out[:N, :C]


# speedup vs baseline: 48.9426x; 1.7499x over previous
import functools

import jax
import jax.numpy as jnp
from jax import lax
from jax.experimental import pallas as pl
from jax.experimental.pallas import tpu as pltpu
from jax.experimental.pallas import tpu_sc as plsc

N = 10000
F_IN = 128
H1 = 32
H2 = 16
C = 40
CPAD = 48
E = 320000

NCORE = 2
NSUB = 16
NWORK = NCORE * NSUB
B = 128
WPT = 80
EPAD = NWORK * WPT * B
NPAD = 10240
RPT = NPAD // NSUB

_BN_SCALE = 1.0 / (1.0 + 1e-5) ** 0.5


def _sc_mesh():
    return plsc.VectorSubcoreMesh(core_axis_name="c", subcore_axis_name="s")


_SC_PARAMS = pltpu.CompilerParams(use_tc_tiling_on_sc=False)


def _make_degree_kernel():
    @functools.partial(
        pl.kernel,
        mesh=_sc_mesh(),
        out_type=(
            jax.ShapeDtypeStruct((NPAD,), jnp.float32),
            jax.ShapeDtypeStruct((NPAD,), jnp.float32),
        ),
        scratch_types=[
            pltpu.VMEM((WPT, B), jnp.int32),
            pltpu.VMEM((B,), jnp.float32),
            pltpu.VMEM_SHARED((NPAD,), jnp.float32),
        ],
        compiler_params=_SC_PARAMS,
    )
    def deg_kernel(col_hbm, zero_hbm, outa, outb, colbuf, ones, acc):
        core = lax.axis_index("c")
        sub = lax.axis_index("s")
        wid = core * NSUB + sub
        r0 = sub * RPT
        pltpu.sync_copy(zero_hbm.at[pl.ds(r0, RPT)], acc.at[pl.ds(r0, RPT)])
        pltpu.sync_copy(col_hbm.at[pl.ds(wid * WPT, WPT)], colbuf)
        for i in range(B // 16):
            ones[pl.ds(i * 16, 16)] = jnp.ones((16,), jnp.float32)
        plsc.subcore_barrier()

        def body(w, carry):
            pltpu.sync_copy(ones, acc.at[colbuf.at[w]], add=True)
            return carry

        lax.fori_loop(0, WPT, body, 0)
        plsc.subcore_barrier()

        @pl.when(core == 0)
        def _():
            pltpu.sync_copy(acc.at[pl.ds(r0, RPT)], outa.at[pl.ds(r0, RPT)])

        @pl.when(core == 1)
        def _():
            pltpu.sync_copy(acc.at[pl.ds(r0, RPT)], outb.at[pl.ds(r0, RPT)])

    return deg_kernel


NGRP = 4
NBUF = 2 * NGRP
NOUT = WPT // (2 * NGRP)


def _make_gather_scatter(h):
    @functools.partial(
        pl.kernel,
        mesh=_sc_mesh(),
        out_type=(
            jax.ShapeDtypeStruct((NPAD, h), jnp.float32),
            jax.ShapeDtypeStruct((NPAD, h), jnp.float32),
        ),
        scratch_types=[
            pltpu.VMEM((WPT, B), jnp.int32),
            pltpu.VMEM((WPT, B), jnp.int32),
            pltpu.VMEM((NBUF * B, h), jnp.float32),
            pltpu.VMEM_SHARED((NPAD, h), jnp.float32),
        ] + [pltpu.SemaphoreType.DMA] * (2 * NBUF),
        compiler_params=_SC_PARAMS,
    )
    def gs_kernel(z_hbm, row_hbm, col_hbm, zero_hbm, outa, outb,
                  rowbuf, colbuf, msg, acc, *sems):
        gsem = sems[:NBUF]
        ssem = sems[NBUF:]
        core = lax.axis_index("c")
        sub = lax.axis_index("s")
        wid = core * NSUB + sub
        r0 = sub * RPT
        pltpu.sync_copy(zero_hbm.at[pl.ds(r0, RPT)], acc.at[pl.ds(r0, RPT)])
        pltpu.sync_copy(row_hbm.at[pl.ds(wid * WPT, WPT)], rowbuf)
        pltpu.sync_copy(col_hbm.at[pl.ds(wid * WPT, WPT)], colbuf)

        def mslice(bi):
            return msg.at[pl.ds(bi * B, B)]

        def g_issue(w, bi):
            pltpu.async_copy(z_hbm.at[rowbuf.at[w]], mslice(bi), gsem[bi])

        def g_wait(w, bi):
            pltpu.make_async_copy(z_hbm.at[rowbuf.at[w]], mslice(bi),
                                  gsem[bi]).wait()

        def s_issue(w, bi):
            pltpu.async_copy(mslice(bi), acc.at[colbuf.at[w]], ssem[bi],
                             add=True)

        def s_wait(bi):
            pltpu.make_async_copy(mslice(bi), acc.at[colbuf.at[0]],
                                  ssem[bi]).wait()

        plsc.subcore_barrier()

        for b in range(NGRP):
            g_issue(b, b)

        def body(k, carry):
            for half in range(2):
                g = 2 * k + half
                my, other = half * NGRP, (1 - half) * NGRP
                for b in range(NGRP):
                    ob = other + b

                    if half == 0:
                        @pl.when(k > 0)
                        def _(ob=ob):
                            s_wait(ob)
                        g_issue((g + 1) * NGRP + b, ob)
                    else:
                        s_wait(ob)

                        @pl.when(k < NOUT - 1)
                        def _(g=g, b=b, ob=ob):
                            g_issue((g + 1) * NGRP + b, ob)
                for b in range(NGRP):
                    w = g * NGRP + b
                    g_wait(w, my + b)
                    s_issue(w, my + b)
            return carry

        lax.fori_loop(0, NOUT, body, 0)
        for b in range(NGRP):
            s_wait(NGRP + b)
        plsc.subcore_barrier()

        @pl.when(core == 0)
        def _():
            pltpu.sync_copy(acc.at[pl.ds(r0, RPT)], outa.at[pl.ds(r0, RPT)])

        @pl.when(core == 1)
        def _():
            pltpu.sync_copy(acc.at[pl.ds(r0, RPT)], outb.at[pl.ds(r0, RPT)])

    return gs_kernel


_deg_kernel = _make_degree_kernel()
_gs32 = _make_gather_scatter(H1)
_gs16 = _make_gather_scatter(H2)
_gs48 = _make_gather_scatter(CPAD)


def _tc_first(x_ref, w_ref, da_ref, db_ref, o_ref):
    dinv = lax.rsqrt(da_ref[...] + db_ref[...] + 1.0)
    xw = jnp.dot(x_ref[...], w_ref[...], preferred_element_type=jnp.float32)
    o_ref[0:N] = xw * dinv[0:N]
    o_ref[N:NPAD] = jnp.zeros((NPAD - N, H1), jnp.float32)


def _tc_mid(sa_ref, sb_ref, z_ref, da_ref, db_ref, b_ref, g_ref, be_ref,
            w_ref, o_ref):
    dinv = lax.rsqrt(da_ref[...] + db_ref[...] + 1.0)
    conv = dinv * (sa_ref[...] + sb_ref[...] + z_ref[...]) + b_ref[...]
    h = jnp.maximum(conv * (g_ref[...] * _BN_SCALE) + be_ref[...], 0.0)
    o_ref[...] = jnp.dot(h, w_ref[...], preferred_element_type=jnp.float32) * dinv


def _tc_last(sa_ref, sb_ref, z_ref, da_ref, db_ref, b_ref, o_ref):
    dinv = lax.rsqrt(da_ref[0:N] + db_ref[0:N] + 1.0)
    tot = sa_ref[0:N, 0:C] + sb_ref[0:N, 0:C] + z_ref[0:N, 0:C]
    o_ref[...] = jnp.maximum(dinv * tot + b_ref[...], 0.0)


def _pc(fn, out_shape):
    return pl.pallas_call(fn, out_shape=out_shape)


def kernel(x, edge_index, W1, b1, g1, be1, W2, b2, g2, be2, W3, b3):
    f32 = jnp.float32
    row = edge_index[0]
    col = edge_index[1]
    pad = EPAD - E
    fill_row = (jnp.arange(pad, dtype=jnp.int32) * 41) % N
    fill_col = N + (jnp.arange(pad, dtype=jnp.int32) % 128)
    rowp = jnp.concatenate([row, fill_row]).reshape(NWORK * WPT, B)
    colp = jnp.concatenate([col, fill_col]).reshape(NWORK * WPT, B)

    W3p = jnp.concatenate([W3, jnp.zeros((H2, CPAD - C), f32)], axis=1)

    zero1 = jnp.zeros((NPAD,), f32)
    zero32 = jnp.zeros((NPAD, H1), f32)
    zero16 = jnp.zeros((NPAD, H2), f32)
    zero48 = jnp.zeros((NPAD, CPAD), f32)

    dega, degb = _deg_kernel(colp, zero1)
    da = dega.reshape(NPAD, 1)
    db = degb.reshape(NPAD, 1)

    z1 = _pc(_tc_first, jax.ShapeDtypeStruct((NPAD, H1), f32))(x, W1, da, db)
    s1a, s1b = _gs32(z1, rowp, colp, zero32)

    z2 = _pc(_tc_mid, jax.ShapeDtypeStruct((NPAD, H2), f32))(
        s1a, s1b, z1, da, db, b1.reshape(1, H1), g1.reshape(1, H1),
        be1.reshape(1, H1), W2)
    s2a, s2b = _gs16(z2, rowp, colp, zero16)

    z3 = _pc(_tc_mid, jax.ShapeDtypeStruct((NPAD, CPAD), f32))(
        s2a, s2b, z2, da, db, b2.reshape(1, H2), g2.reshape(1, H2),
        be2.reshape(1, H2), W3p)
    s3a, s3b = _gs48(z3, rowp, colp, zero48)

    return _pc(_tc_last, jax.ShapeDtypeStruct((N, C), f32))(
        s3a, s3b, z3, da, db, b3.reshape(1, C))


# speedup vs baseline: 49.6626x; 1.0147x over previous
import functools

import jax
import jax.numpy as jnp
from jax import lax
from jax.experimental import pallas as pl
from jax.experimental.pallas import tpu as pltpu
from jax.experimental.pallas import tpu_sc as plsc

N = 10000
F_IN = 128
H1 = 32
H2 = 16
C = 40
CPAD = 48
E = 320000

NCORE = 2
NSUB = 16
NWORK = NCORE * NSUB
B = 128
WPT = 80
EPAD = NWORK * WPT * B
NPAD = 10240
RPT = NPAD // NSUB

_BN_SCALE = 1.0 / (1.0 + 1e-5) ** 0.5


def _sc_mesh():
    return plsc.VectorSubcoreMesh(core_axis_name="c", subcore_axis_name="s")


_SC_PARAMS = pltpu.CompilerParams(use_tc_tiling_on_sc=False)


def _make_degree_kernel():
    @functools.partial(
        pl.kernel,
        mesh=_sc_mesh(),
        out_type=(
            jax.ShapeDtypeStruct((NPAD,), jnp.float32),
            jax.ShapeDtypeStruct((NPAD,), jnp.float32),
        ),
        scratch_types=[
            pltpu.VMEM((WPT, B), jnp.int32),
            pltpu.VMEM((B,), jnp.float32),
            pltpu.VMEM_SHARED((NPAD,), jnp.float32),
        ],
        compiler_params=_SC_PARAMS,
    )
    def deg_kernel(col_hbm, zero_hbm, outa, outb, colbuf, ones, acc):
        core = lax.axis_index("c")
        sub = lax.axis_index("s")
        wid = core * NSUB + sub
        r0 = sub * RPT
        pltpu.sync_copy(zero_hbm.at[pl.ds(r0, RPT)], acc.at[pl.ds(r0, RPT)])
        pltpu.sync_copy(col_hbm.at[pl.ds(wid * WPT, WPT)], colbuf)
        for i in range(B // 16):
            ones[pl.ds(i * 16, 16)] = jnp.ones((16,), jnp.float32)
        plsc.subcore_barrier()

        def body(w, carry):
            pltpu.sync_copy(ones, acc.at[colbuf.at[w]], add=True)
            return carry

        lax.fori_loop(0, WPT, body, 0)
        plsc.subcore_barrier()

        @pl.when(core == 0)
        def _():
            pltpu.sync_copy(acc.at[pl.ds(r0, RPT)], outa.at[pl.ds(r0, RPT)])

        @pl.when(core == 1)
        def _():
            pltpu.sync_copy(acc.at[pl.ds(r0, RPT)], outb.at[pl.ds(r0, RPT)])

    return deg_kernel


NGRP = 4
NBUF = 2 * NGRP
NOUT = WPT // (2 * NGRP)


def _make_gather_scatter(h):
    @functools.partial(
        pl.kernel,
        mesh=_sc_mesh(),
        out_type=(
            jax.ShapeDtypeStruct((NPAD, h), jnp.float32),
            jax.ShapeDtypeStruct((NPAD, h), jnp.float32),
        ),
        scratch_types=[
            pltpu.VMEM((WPT, B), jnp.int32),
            pltpu.VMEM((WPT, B), jnp.int32),
            pltpu.VMEM((NBUF * B, h), jnp.float32),
            pltpu.VMEM_SHARED((NPAD, h), jnp.float32),
        ] + [pltpu.SemaphoreType.DMA] * (2 * NBUF),
        compiler_params=_SC_PARAMS,
    )
    def gs_kernel(z_hbm, row_hbm, col_hbm, zero_hbm, outa, outb,
                  rowbuf, colbuf, msg, acc, *sems):
        gsem = sems[:NBUF]
        ssem = sems[NBUF:]
        core = lax.axis_index("c")
        sub = lax.axis_index("s")
        wid = core * NSUB + sub
        r0 = sub * RPT
        pltpu.sync_copy(zero_hbm.at[pl.ds(r0, RPT)], acc.at[pl.ds(r0, RPT)])
        pltpu.sync_copy(row_hbm.at[pl.ds(wid * WPT, WPT)], rowbuf)
        pltpu.sync_copy(col_hbm.at[pl.ds(wid * WPT, WPT)], colbuf)

        def mslice(bi):
            return msg.at[pl.ds(bi * B, B)]

        def g_issue(w, bi):
            pltpu.async_copy(z_hbm.at[rowbuf.at[w]], mslice(bi), gsem[bi])

        def g_wait(w, bi):
            pltpu.make_async_copy(z_hbm.at[rowbuf.at[w]], mslice(bi),
                                  gsem[bi]).wait()

        def s_issue(w, bi):
            pltpu.async_copy(mslice(bi), acc.at[colbuf.at[w]], ssem[bi],
                             add=True)

        def s_wait(bi):
            pltpu.make_async_copy(mslice(bi), acc.at[colbuf.at[0]],
                                  ssem[bi]).wait()

        plsc.subcore_barrier()

        for b in range(NGRP):
            g_issue(b, b)

        def body(k, carry):
            for half in range(2):
                g = 2 * k + half
                my, other = half * NGRP, (1 - half) * NGRP
                for b in range(NGRP):
                    ob = other + b

                    if half == 0:
                        @pl.when(k > 0)
                        def _(ob=ob):
                            s_wait(ob)
                        g_issue((g + 1) * NGRP + b, ob)
                    else:
                        s_wait(ob)

                        @pl.when(k < NOUT - 1)
                        def _(g=g, b=b, ob=ob):
                            g_issue((g + 1) * NGRP + b, ob)
                for b in range(NGRP):
                    w = g * NGRP + b
                    g_wait(w, my + b)
                    s_issue(w, my + b)
            return carry

        lax.fori_loop(0, NOUT, body, 0)
        for b in range(NGRP):
            s_wait(NGRP + b)
        plsc.subcore_barrier()

        @pl.when(core == 0)
        def _():
            pltpu.sync_copy(acc.at[pl.ds(r0, RPT)], outa.at[pl.ds(r0, RPT)])

        @pl.when(core == 1)
        def _():
            pltpu.sync_copy(acc.at[pl.ds(r0, RPT)], outb.at[pl.ds(r0, RPT)])

    return gs_kernel


_deg_kernel = _make_degree_kernel()
_gs32 = _make_gather_scatter(H1)
_gs16 = _make_gather_scatter(H2)
_gs48 = _make_gather_scatter(CPAD)


_BLK = 1280


def _tc_first(x_ref, w_ref, da_ref, db_ref, z_ref, d48_ref):
    i = pl.program_id(0)
    dinv = lax.rsqrt(da_ref[...] + db_ref[...] + 1.0)
    d48 = jnp.broadcast_to(dinv, (_BLK, CPAD))
    d48_ref[...] = d48
    xw = jnp.dot(x_ref[...], w_ref[...], preferred_element_type=jnp.float32)
    rows = lax.broadcasted_iota(jnp.int32, (_BLK, 1), 0) + i * _BLK
    z_ref[...] = jnp.where(rows < N, xw * d48[:, :H1], 0.0)


def _make_tc_first():
    full = lambda s: pl.BlockSpec(s, lambda i: (0, 0))
    blk = lambda w: pl.BlockSpec((_BLK, w), lambda i: (i, 0))
    return pl.pallas_call(
        _tc_first,
        grid=(NPAD // _BLK,),
        in_specs=[blk(F_IN), full((F_IN, H1)), blk(1), blk(1)],
        out_specs=[blk(H1), blk(CPAD)],
        out_shape=[jax.ShapeDtypeStruct((NPAD, H1), jnp.float32),
                   jax.ShapeDtypeStruct((NPAD, CPAD), jnp.float32)],
    )


def _make_tc_mid(h_in, h_out):
    def body(sa_ref, sb_ref, z_ref, d48_ref, b_ref, g_ref, be_ref, w_ref,
             o_ref):
        d48 = d48_ref[...]
        conv = d48[:, :h_in] * (sa_ref[...] + sb_ref[...] + z_ref[...]) \
            + b_ref[...]
        h = jnp.maximum(conv * (g_ref[...] * _BN_SCALE) + be_ref[...], 0.0)
        o_ref[...] = jnp.dot(h, w_ref[...],
                             preferred_element_type=jnp.float32) \
            * d48[:, :h_out]

    full = lambda s: pl.BlockSpec(s, lambda i: (0, 0))
    blk = lambda w: pl.BlockSpec((_BLK, w), lambda i: (i, 0))
    return pl.pallas_call(
        body,
        grid=(NPAD // _BLK,),
        in_specs=[blk(h_in), blk(h_in), blk(h_in), blk(CPAD), full((1, h_in)),
                  full((1, h_in)), full((1, h_in)), full((h_in, h_out))],
        out_specs=blk(h_out),
        out_shape=jax.ShapeDtypeStruct((NPAD, h_out), jnp.float32),
    )


_OBLK = 1000


def _tc_last(sa_ref, sb_ref, z_ref, d48_ref, b_ref, o_ref):
    tot = sa_ref[...] + sb_ref[...] + z_ref[...]
    o_ref[...] = jnp.maximum(d48_ref[...][:, :C] * tot[:, :C] + b_ref[...],
                             0.0)


def _make_tc_last():
    full = lambda s: pl.BlockSpec(s, lambda i: (0, 0))
    blk = lambda w: pl.BlockSpec((_OBLK, w), lambda i: (i, 0))
    return pl.pallas_call(
        _tc_last,
        grid=(N // _OBLK,),
        in_specs=[blk(CPAD), blk(CPAD), blk(CPAD), blk(CPAD), full((1, C))],
        out_specs=blk(C),
        out_shape=jax.ShapeDtypeStruct((N, C), jnp.float32),
    )


_tc_first_call = _make_tc_first()
_tc_mid12 = _make_tc_mid(H1, H2)
_tc_mid23 = _make_tc_mid(H2, CPAD)
_tc_last_call = _make_tc_last()


def kernel(x, edge_index, W1, b1, g1, be1, W2, b2, g2, be2, W3, b3):
    f32 = jnp.float32
    row = edge_index[0]
    col = edge_index[1]
    pad = EPAD - E
    fill_row = (jnp.arange(pad, dtype=jnp.int32) * 41) % N
    fill_col = N + (jnp.arange(pad, dtype=jnp.int32) % 128)
    rowp = jnp.concatenate([row, fill_row]).reshape(NWORK * WPT, B)
    colp = jnp.concatenate([col, fill_col]).reshape(NWORK * WPT, B)

    W3p = jnp.concatenate([W3, jnp.zeros((H2, CPAD - C), f32)], axis=1)

    zero1 = jnp.zeros((NPAD,), f32)
    zero32 = jnp.zeros((NPAD, H1), f32)
    zero16 = jnp.zeros((NPAD, H2), f32)
    zero48 = jnp.zeros((NPAD, CPAD), f32)

    dega, degb = _deg_kernel(colp, zero1)
    da = dega.reshape(NPAD, 1)
    db = degb.reshape(NPAD, 1)

    z1, d48 = _tc_first_call(x, W1, da, db)
    s1a, s1b = _gs32(z1, rowp, colp, zero32)

    z2 = _tc_mid12(s1a, s1b, z1, d48, b1.reshape(1, H1), g1.reshape(1, H1),
                   be1.reshape(1, H1), W2)
    s2a, s2b = _gs16(z2, rowp, colp, zero16)

    z3 = _tc_mid23(s2a, s2b, z2, d48, b2.reshape(1, H2), g2.reshape(1, H2),
                   be2.reshape(1, H2), W3p)
    s3a, s3b = _gs48(z3, rowp, colp, zero48)

    return _tc_last_call(s3a, s3b, z3, d48, b3.reshape(1, C))


# speedup vs baseline: 53.8154x; 1.0836x over previous
import functools

import jax
import jax.numpy as jnp
from jax import lax
from jax.experimental import pallas as pl
from jax.experimental.pallas import tpu as pltpu
from jax.experimental.pallas import tpu_sc as plsc

N = 10000
F_IN = 128
H1 = 32
H2 = 16
C = 40
CPAD = 64
E = 320000

NCORE = 2
NSUB = 16
NWORK = NCORE * NSUB
B = 125
NWIN = E // B
WPT = NWIN // NWORK
NPAD = 10240
RPT = NPAD // NSUB

_BN_SCALE = 1.0 / (1.0 + 1e-5) ** 0.5


def _sc_mesh():
    return plsc.VectorSubcoreMesh(core_axis_name="c", subcore_axis_name="s")


_SC_PARAMS = pltpu.CompilerParams(use_tc_tiling_on_sc=False)


def _make_degree_kernel():
    @functools.partial(
        pl.kernel,
        mesh=_sc_mesh(),
        out_type=(
            jax.ShapeDtypeStruct((NPAD,), jnp.float32),
            jax.ShapeDtypeStruct((NPAD,), jnp.float32),
        ),
        scratch_types=[
            pltpu.VMEM((WPT, B), jnp.int32),
            pltpu.VMEM((128,), jnp.float32),
            pltpu.VMEM_SHARED((NPAD,), jnp.float32),
        ],
        compiler_params=_SC_PARAMS,
    )
    def deg_kernel(idx_hbm, zero_hbm, outa, outb, colbuf, ones, acc):
        core = lax.axis_index("c")
        sub = lax.axis_index("s")
        wid = core * NSUB + sub
        r0 = sub * RPT
        pltpu.sync_copy(zero_hbm.at[pl.ds(r0, RPT)], acc.at[pl.ds(r0, RPT)])
        pltpu.sync_copy(idx_hbm.at[1, pl.ds(wid * WPT, WPT)], colbuf)
        for i in range(128 // 16):
            ones[pl.ds(i * 16, 16)] = jnp.ones((16,), jnp.float32)
        plsc.subcore_barrier()

        def body(w, carry):
            pltpu.sync_copy(ones.at[pl.ds(0, B)], acc.at[colbuf.at[w]],
                            add=True)
            return carry

        lax.fori_loop(0, WPT, body, 0)
        plsc.subcore_barrier()

        @pl.when(core == 0)
        def _():
            pltpu.sync_copy(acc.at[pl.ds(r0, RPT)], outa.at[pl.ds(r0, RPT)])

        @pl.when(core == 1)
        def _():
            pltpu.sync_copy(acc.at[pl.ds(r0, RPT)], outb.at[pl.ds(r0, RPT)])

    return deg_kernel


NGRP = 4
NBUF = 2 * NGRP
NOUT = WPT // (2 * NGRP)


def _make_gather_scatter(h):
    @functools.partial(
        pl.kernel,
        mesh=_sc_mesh(),
        out_type=(
            jax.ShapeDtypeStruct((NPAD, h), jnp.float32),
            jax.ShapeDtypeStruct((NPAD, h), jnp.float32),
        ),
        scratch_types=[
            pltpu.VMEM((WPT, B), jnp.int32),
            pltpu.VMEM((WPT, B), jnp.int32),
            pltpu.VMEM((NBUF * B, h), jnp.float32),
            pltpu.VMEM_SHARED((NPAD, h), jnp.float32),
        ] + [pltpu.SemaphoreType.DMA] * (2 * NBUF),
        compiler_params=_SC_PARAMS,
    )
    def gs_kernel(z_hbm, idx_hbm, zero_hbm, outa, outb,
                  rowbuf, colbuf, msg, acc, *sems):
        gsem = sems[:NBUF]
        ssem = sems[NBUF:]
        core = lax.axis_index("c")
        sub = lax.axis_index("s")
        wid = core * NSUB + sub
        r0 = sub * RPT
        pltpu.sync_copy(zero_hbm.at[pl.ds(r0, RPT)], acc.at[pl.ds(r0, RPT)])
        pltpu.sync_copy(idx_hbm.at[0, pl.ds(wid * WPT, WPT)], rowbuf)
        pltpu.sync_copy(idx_hbm.at[1, pl.ds(wid * WPT, WPT)], colbuf)

        def mslice(bi):
            return msg.at[pl.ds(bi * B, B)]

        def g_issue(w, bi):
            pltpu.async_copy(z_hbm.at[rowbuf.at[w]], mslice(bi), gsem[bi])

        def g_wait(w, bi):
            pltpu.make_async_copy(z_hbm.at[rowbuf.at[w]], mslice(bi),
                                  gsem[bi]).wait()

        def s_issue(w, bi):
            pltpu.async_copy(mslice(bi), acc.at[colbuf.at[w]], ssem[bi],
                             add=True)

        def s_wait(bi):
            pltpu.make_async_copy(mslice(bi), acc.at[colbuf.at[0]],
                                  ssem[bi]).wait()

        plsc.subcore_barrier()

        for b in range(NGRP):
            g_issue(b, b)

        def body(k, carry):
            for half in range(2):
                g = 2 * k + half
                my, other = half * NGRP, (1 - half) * NGRP
                for b in range(NGRP):
                    ob = other + b

                    if half == 0:
                        @pl.when(k > 0)
                        def _(ob=ob):
                            s_wait(ob)
                        g_issue((g + 1) * NGRP + b, ob)
                    else:
                        s_wait(ob)

                        @pl.when(k < NOUT - 1)
                        def _(g=g, b=b, ob=ob):
                            g_issue((g + 1) * NGRP + b, ob)
                for b in range(NGRP):
                    w = g * NGRP + b
                    g_wait(w, my + b)
                    s_issue(w, my + b)
            return carry

        lax.fori_loop(0, NOUT, body, 0)
        for b in range(NGRP):
            s_wait(NGRP + b)
        plsc.subcore_barrier()

        @pl.when(core == 0)
        def _():
            pltpu.sync_copy(acc.at[pl.ds(r0, RPT)], outa.at[pl.ds(r0, RPT)])

        @pl.when(core == 1)
        def _():
            pltpu.sync_copy(acc.at[pl.ds(r0, RPT)], outb.at[pl.ds(r0, RPT)])

    return gs_kernel


_deg_kernel = _make_degree_kernel()
_gs32 = _make_gather_scatter(H1)
_gs16 = _make_gather_scatter(H2)
_gs64 = _make_gather_scatter(CPAD)


_BLK = 2048
_GRID = NPAD // _BLK


def _unpack(xp, h):
    p = 128 // h
    rows = xp.shape[0]
    parts = [xp[:, h * j:h * (j + 1)] for j in range(p)]
    return jnp.stack(parts, axis=1).reshape(rows * p, h)


def _pack_matmul(hmat, w, dcol, h_out):
    p = 128 // h_out
    n = hmat.shape[0]
    h3 = hmat.reshape(n // p, p, hmat.shape[1])
    d3 = dcol.reshape(n // p, p, 1)
    outs = [jnp.dot(h3[:, j, :], w, preferred_element_type=jnp.float32)
            * d3[:, j, :] for j in range(p)]
    return jnp.concatenate(outs, axis=1)


def _tc_first(x_ref, w_ref, da_ref, db_ref, z_ref, d_ref):
    i = pl.program_id(0)
    dflat = lax.rsqrt(da_ref[...] + db_ref[...] + 1.0)
    r_sub = lax.broadcasted_iota(jnp.int32, (_BLK, 16), 0) // 128
    c_sub = lax.broadcasted_iota(jnp.int32, (_BLK, 16), 1)
    e_sel = (r_sub == c_sub).astype(jnp.float32)
    t1 = jnp.dot(e_sel, dflat, preferred_element_type=jnp.float32)
    lane = lax.broadcasted_iota(jnp.int32, (_BLK, 128), 1)
    ridx = lax.broadcasted_iota(jnp.int32, (_BLK, 128), 0)
    dcol = jnp.sum(jnp.where(lane == ridx % 128, t1, 0.0), axis=1,
                   keepdims=True)
    d_ref[...] = dcol
    x = x_ref[...]
    rows = lax.broadcasted_iota(jnp.int32, (_BLK, 1), 0) + i * _BLK
    xm = jnp.where(rows < N, x, 0.0)
    z_ref[...] = _pack_matmul(xm, w_ref[...], dcol, H1)


def _make_tc_first():
    full = lambda s: pl.BlockSpec(s, lambda i: (0, 0))
    return pl.pallas_call(
        _tc_first,
        grid=(_GRID,),
        in_specs=[pl.BlockSpec((_BLK, F_IN), lambda i: (i, 0)),
                  full((F_IN, H1)),
                  pl.BlockSpec((16, 128), lambda i: (i, 0)),
                  pl.BlockSpec((16, 128), lambda i: (i, 0))],
        out_specs=[pl.BlockSpec((_BLK * H1 // 128, 128), lambda i: (i, 0)),
                   pl.BlockSpec((_BLK, 1), lambda i: (i, 0))],
        out_shape=[jax.ShapeDtypeStruct((NPAD * H1 // 128, 128), jnp.float32),
                   jax.ShapeDtypeStruct((NPAD, 1), jnp.float32)],
    )


def _make_tc_mid(h_in, h_out):
    pr_in = _BLK * h_in // 128

    def body(sa_ref, sb_ref, z_ref, d_ref, b_ref, g_ref, be_ref, w_ref,
             o_ref):
        tot = _unpack(sa_ref[...] + sb_ref[...] + z_ref[...], h_in)
        dcol = d_ref[...]
        conv = dcol * tot + b_ref[...]
        h = jnp.maximum(conv * (g_ref[...] * _BN_SCALE) + be_ref[...], 0.0)
        o_ref[...] = _pack_matmul(h, w_ref[...], dcol, h_out)

    full = lambda s: pl.BlockSpec(s, lambda i: (0, 0))
    pblk = lambda r: pl.BlockSpec((r, 128), lambda i: (i, 0))
    return pl.pallas_call(
        body,
        grid=(_GRID,),
        in_specs=[pblk(pr_in), pblk(pr_in), pblk(pr_in),
                  pl.BlockSpec((_BLK, 1), lambda i: (i, 0)),
                  full((1, h_in)), full((1, h_in)), full((1, h_in)),
                  full((h_in, h_out))],
        out_specs=pblk(_BLK * h_out // 128),
        out_shape=jax.ShapeDtypeStruct((NPAD * h_out // 128, 128),
                                       jnp.float32),
    )


_OBLK = 2000


def _tc_last(sa_ref, sb_ref, z_ref, d_ref, b_ref, o_ref):
    tot = _unpack(sa_ref[...] + sb_ref[...] + z_ref[...], CPAD)
    o_ref[...] = jnp.maximum(d_ref[...] * tot[:, :C] + b_ref[...], 0.0)


def _make_tc_last():
    full = lambda s: pl.BlockSpec(s, lambda i: (0, 0))
    pblk = pl.BlockSpec((_OBLK * CPAD // 128, 128), lambda i: (i, 0))
    return pl.pallas_call(
        _tc_last,
        grid=(N // _OBLK,),
        in_specs=[pblk, pblk, pblk,
                  pl.BlockSpec((_OBLK, 1), lambda i: (i, 0)),
                  full((1, C))],
        out_specs=pl.BlockSpec((_OBLK, C), lambda i: (i, 0)),
        out_shape=jax.ShapeDtypeStruct((N, C), jnp.float32),
    )


_tc_first_call = _make_tc_first()
_tc_mid12 = _make_tc_mid(H1, H2)
_tc_mid23 = _make_tc_mid(H2, CPAD)
_tc_last_call = _make_tc_last()


def kernel(x, edge_index, W1, b1, g1, be1, W2, b2, g2, be2, W3, b3):
    f32 = jnp.float32
    idx3 = edge_index.reshape(2, NWIN, B)

    W3p = jnp.concatenate([W3, jnp.zeros((H2, CPAD - C), f32)], axis=1)

    zero1 = jnp.zeros((NPAD,), f32)
    zero32 = jnp.zeros((NPAD, H1), f32)
    zero16 = jnp.zeros((NPAD, H2), f32)
    zero64 = jnp.zeros((NPAD, CPAD), f32)

    dega, degb = _deg_kernel(idx3, zero1)

    z1p, dinv = _tc_first_call(x, W1, dega.reshape(NPAD // 128, 128),
                               degb.reshape(NPAD // 128, 128))
    s1a, s1b = _gs32(z1p.reshape(NPAD, H1), idx3, zero32)

    z2p = _tc_mid12(s1a.reshape(NPAD * H1 // 128, 128),
                    s1b.reshape(NPAD * H1 // 128, 128), z1p, dinv,
                    b1.reshape(1, H1), g1.reshape(1, H1), be1.reshape(1, H1),
                    W2)
    s2a, s2b = _gs16(z2p.reshape(NPAD, H2), idx3, zero16)

    z3p = _tc_mid23(s2a.reshape(NPAD * H2 // 128, 128),
                    s2b.reshape(NPAD * H2 // 128, 128), z2p, dinv,
                    b2.reshape(1, H2), g2.reshape(1, H2), be2.reshape(1, H2),
                    W3p)
    s3a, s3b = _gs64(z3p.reshape(NPAD, CPAD), idx3, zero64)

    return _tc_last_call(s3a.reshape(NPAD * CPAD // 128, 128),
                         s3b.reshape(NPAD * CPAD // 128, 128), z3p, dinv,
                         b3.reshape(1, C))
